# Initial kernel scaffold; baseline (speedup 1.0000x reference)
import jax
import jax.numpy as jnp
from jax.experimental import pallas as pl


def kernel(other_ids, seq_flat_ids, cu_seqlens, target_ids, wide_ids, deep_ids, din_table, aW1, ab1, aa1, aW2, ab2, aa2, aWo, abo, mW1, mb1, ma1, mW2, mb2, ma2, mWo, mbo, wide_table, lrW, lrb, deep_table, dW1, db1, dW2, db2, dWo, dbo):
    raise NotImplementedError("write your pallas kernel here")


# speedup vs baseline: 10.5078x; 10.5078x over previous
import functools

import jax
import jax.numpy as jnp
from jax import lax
from jax.experimental import pallas as pl
from jax.experimental.pallas import tpu as pltpu
from jax.experimental.pallas import tpu_sc as plsc

NC = 2
NS = 16
NW = NC * NS


def _sc_gather(addr, seq_flat_ids, din_table, deep_table, wide_table,
               deep_idx, wide_idx, oth_idx, tgt_idx,
               BL, D, WD, FPAD):
    SEQ_CH = addr.shape[1]
    F_CH = deep_idx.shape[1]
    BPW = oth_idx.shape[1]
    B = NW * BPW
    SEQ_PW = SEQ_CH * 128

    mesh = plsc.VectorSubcoreMesh(core_axis_name="c", subcore_axis_name="s")

    @functools.partial(
        pl.kernel,
        out_type=(
            jax.ShapeDtypeStruct((BL, D), jnp.float32),
            jax.ShapeDtypeStruct((FPAD, D), jnp.float32),
            jax.ShapeDtypeStruct((FPAD, WD), jnp.float32),
            jax.ShapeDtypeStruct((B, D), jnp.float32),
            jax.ShapeDtypeStruct((B, D), jnp.float32),
        ),
        mesh=mesh,
        compiler_params=pltpu.CompilerParams(use_tc_tiling_on_sc=False),
        scratch_types=[
            pltpu.VMEM((SEQ_CH, 128), jnp.int32),
            pltpu.VMEM((SEQ_CH, 128), jnp.int32),
            pltpu.VMEM((F_CH, 128), jnp.int32),
            pltpu.VMEM((2, BPW), jnp.int32),
            pltpu.VMEM((128, D), jnp.float32),
            pltpu.VMEM((128, WD), jnp.float32),
            pltpu.VMEM((BPW, D), jnp.float32),
            pltpu.SemaphoreType.DMA,
        ],
    )
    def k(addr_hbm, seqids_hbm, dinT_hbm, deepT_hbm, wideT_hbm,
          didx_hbm, widx_hbm, oidx_hbm, tidx_hbm,
          padded_hbm, drows_hbm, wrows_hbm, oemb_hbm, temb_hbm,
          addr_v, ids_v, fidx_v, smidx_v, rows_v, wrows_v, srows_v, sem):
        c = lax.axis_index("c")
        s = lax.axis_index("s")
        wid = s * NC + c

        pltpu.sync_copy(addr_hbm.at[wid], addr_v)

        def g1(j, carry):
            pltpu.async_copy(seqids_hbm.at[addr_v.at[j]], ids_v.at[j],
                             sem).wait()
            return carry

        lax.fori_loop(0, SEQ_CH, g1, 0)

        sbase = wid * SEQ_PW

        def g2(j, carry):
            pltpu.async_copy(dinT_hbm.at[ids_v.at[j]], rows_v, sem).wait()
            pltpu.sync_copy(rows_v, padded_hbm.at[pl.ds(sbase + j * 128, 128)])
            return carry

        lax.fori_loop(0, SEQ_CH, g2, 0)

        pltpu.sync_copy(didx_hbm.at[wid], fidx_v)
        fbase = wid * F_CH * 128

        def g3(j, carry):
            pltpu.async_copy(deepT_hbm.at[fidx_v.at[j]], rows_v, sem).wait()
            pltpu.sync_copy(rows_v, drows_hbm.at[pl.ds(fbase + j * 128, 128)])
            return carry

        lax.fori_loop(0, F_CH, g3, 0)

        pltpu.sync_copy(widx_hbm.at[wid], fidx_v)

        def g4(j, carry):
            pltpu.async_copy(wideT_hbm.at[fidx_v.at[j]], wrows_v, sem).wait()
            pltpu.sync_copy(wrows_v, wrows_hbm.at[pl.ds(fbase + j * 128, 128)])
            return carry

        lax.fori_loop(0, F_CH, g4, 0)

        pltpu.sync_copy(oidx_hbm.at[wid], smidx_v.at[0])
        pltpu.sync_copy(tidx_hbm.at[wid], smidx_v.at[1])
        bbase = wid * BPW
        pltpu.async_copy(dinT_hbm.at[smidx_v.at[0]], srows_v, sem).wait()
        pltpu.sync_copy(srows_v, oemb_hbm.at[pl.ds(bbase, BPW)])
        pltpu.async_copy(dinT_hbm.at[smidx_v.at[1]], srows_v, sem).wait()
        pltpu.sync_copy(srows_v, temb_hbm.at[pl.ds(bbase, BPW)])

    return k(addr, seq_flat_ids, din_table, deep_table, wide_table,
             deep_idx, wide_idx, oth_idx, tgt_idx)


def _dice(x, st_ref, n, alpha_ref):
    st = st_ref[...]
    mean = st[0:1, :] * (1.0 / n)
    var = st[1:2, :] * (1.0 / n) - mean * mean
    xn = (x - mean) * lax.rsqrt(var + 1e-5)
    p = jax.nn.sigmoid(xn)
    return x * (p + (1.0 - p) * alpha_ref[...])


def _p1_body(BB, LMAX, D, pad_ref, tgt_ref, len_ref, w_ref, b_ref,
             h_ref, st_ref):
    i = pl.program_id(0)
    R = BB * LMAX
    p3 = pad_ref[...].reshape(BB, LMAX, D)
    liota = lax.broadcasted_iota(jnp.int32, (BB, LMAX, D), 1).astype(jnp.float32)
    len3 = len_ref[...].reshape(BB, 1, 1)
    p3 = jnp.where(liota < len3, p3, 0.0)
    t3 = jnp.broadcast_to(tgt_ref[...].reshape(BB, 1, D), (BB, LMAX, D))
    att = jnp.concatenate([t3, p3, t3 - p3, t3 * p3], axis=2)
    att = att.reshape(R, 4 * D)
    h = jnp.dot(att, w_ref[...], preferred_element_type=jnp.float32)
    h = h + b_ref[...]
    h_ref[...] = h

    @pl.when(i == 0)
    def _():
        st_ref[...] = jnp.zeros_like(st_ref)

    st_ref[0:1, :] += jnp.sum(h, axis=0, keepdims=True)
    st_ref[1:2, :] += jnp.sum(h * h, axis=0, keepdims=True)


def _p2_body(n1, h1_ref, st1_ref, a1_ref, w_ref, b_ref, h2_ref, st2_ref):
    i = pl.program_id(0)
    x = _dice(h1_ref[...], st1_ref, n1, a1_ref)
    h2 = jnp.dot(x, w_ref[...], preferred_element_type=jnp.float32)
    h2 = h2 + b_ref[...]
    h2_ref[...] = h2

    @pl.when(i == 0)
    def _():
        st2_ref[...] = jnp.zeros_like(st2_ref)

    st2_ref[0:1, :] += jnp.sum(h2, axis=0, keepdims=True)
    st2_ref[1:2, :] += jnp.sum(h2 * h2, axis=0, keepdims=True)


def _p3_body(n1, BB, LMAX, D, h2_ref, st2_ref, a2_ref, wo_ref, bo_ref,
             pad_ref, len_ref, pooled_ref):
    x = _dice(h2_ref[...], st2_ref, n1, a2_ref)
    s = jnp.sum(x * wo_ref[...], axis=1, keepdims=True) + bo_ref[...]
    s3 = s.reshape(BB, LMAX, 1)
    liota = lax.broadcasted_iota(jnp.int32, (BB, LMAX, 1), 1).astype(jnp.float32)
    len3 = len_ref[...].reshape(BB, 1, 1)
    s3 = jnp.where(liota < len3, s3, 0.0)
    p3 = pad_ref[...].reshape(BB, LMAX, D)
    pooled_ref[...] = jnp.sum(p3 * s3, axis=1)


def _bn_in(x, n):
    mean = jnp.sum(x, axis=0, keepdims=True) * (1.0 / n)
    d = x - mean
    var = jnp.sum(d * d, axis=0, keepdims=True) * (1.0 / n)
    return d * lax.rsqrt(var + 1e-5)


def _dice_in(x, n, alpha):
    xn = _bn_in(x, n)
    p = jax.nn.sigmoid(xn)
    return x * (p + (1.0 - p) * alpha)


def _p4_body(B,
             oth_ref, pool_ref, tgt_ref,
             mW1_ref, mb1_ref, ma1_ref, mW2_ref, mb2_ref, ma2_ref,
             mWo_ref, mbo_ref,
             wide_ref, lrW_ref, lrb_ref,
             deep_ref, dW1_ref, db1_ref, dW2_ref, db2_ref, dWo_ref, dbo_ref,
             out_ref):
    n = float(B)
    emb = jnp.concatenate([oth_ref[...], pool_ref[...], tgt_ref[...]], axis=1)
    x = jnp.dot(emb, mW1_ref[...], preferred_element_type=jnp.float32)
    x = _dice_in(x + mb1_ref[...], n, ma1_ref[...])
    x = jnp.dot(x, mW2_ref[...], preferred_element_type=jnp.float32)
    x = _dice_in(x + mb2_ref[...], n, ma2_ref[...])
    din = jnp.sum(x * mWo_ref[...], axis=1, keepdims=True) + mbo_ref[...]

    lr = jnp.sum(wide_ref[...] * lrW_ref[...], axis=1, keepdims=True)
    lr = lr + lrb_ref[...]

    d = jnp.dot(deep_ref[...], dW1_ref[...], preferred_element_type=jnp.float32)
    d = jax.nn.relu(_bn_in(d + db1_ref[...], n))
    d = jnp.dot(d, dW2_ref[...], preferred_element_type=jnp.float32)
    d = jax.nn.relu(_bn_in(d + db2_ref[...], n))
    deep = jnp.sum(d * dWo_ref[...], axis=1, keepdims=True) + dbo_ref[...]

    out_ref[...] = jax.nn.sigmoid(din + lr + deep)


def kernel(other_ids, seq_flat_ids, cu_seqlens, target_ids, wide_ids, deep_ids,
           din_table, aW1, ab1, aa1, aW2, ab2, aa2, aWo, abo,
           mW1, mb1, ma1, mW2, mb2, ma2, mWo, mbo,
           wide_table, lrW, lrb, deep_table, dW1, db1, dW2, db2, dWo, dbo):
    B = other_ids.shape[0]
    T = seq_flat_ids.shape[0]
    D = din_table.shape[1]
    WD = wide_table.shape[1]
    NF = wide_ids.shape[1]
    LMAX = 200
    BL = B * LMAX

    cu = cu_seqlens.astype(jnp.int32)
    lengths = cu[1:] - cu[:-1]
    addr = cu[:-1, None] + jnp.arange(LMAX, dtype=jnp.int32)[None, :]
    addr = jnp.minimum(addr, T - 1).reshape(NW, BL // NW // 128, 128)

    nf_flat = B * NF
    F_CH = -(-nf_flat // (NW * 128))
    FPAD = NW * F_CH * 128
    zpad = jnp.zeros((FPAD - nf_flat,), jnp.int32)
    didx = jnp.concatenate([deep_ids.reshape(-1).astype(jnp.int32), zpad])
    widx = jnp.concatenate([wide_ids.reshape(-1).astype(jnp.int32), zpad])
    didx = didx.reshape(NW, F_CH, 128)
    widx = widx.reshape(NW, F_CH, 128)
    oidx = other_ids.astype(jnp.int32).reshape(NW, B // NW)
    tidx = target_ids.astype(jnp.int32).reshape(NW, B // NW)

    padded, drows, wrows, oth_emb, tgt_emb = _sc_gather(
        addr, seq_flat_ids.astype(jnp.int32), din_table, deep_table,
        wide_table, didx, widx, oidx, tidx, BL, D, WD, FPAD)

    lenf = lengths.astype(jnp.float32).reshape(B, 1)

    BB = 8
    G1 = B // BB
    R = BB * LMAX
    H1 = aW1.shape[1]
    h1, st1 = pl.pallas_call(
        functools.partial(_p1_body, BB, LMAX, D),
        grid=(G1,),
        in_specs=[
            pl.BlockSpec((R, D), lambda i: (i, 0)),
            pl.BlockSpec((BB, D), lambda i: (i, 0)),
            pl.BlockSpec((BB, 1), lambda i: (i, 0)),
            pl.BlockSpec((4 * D, H1), lambda i: (0, 0)),
            pl.BlockSpec((1, H1), lambda i: (0, 0)),
        ],
        out_specs=[
            pl.BlockSpec((R, H1), lambda i: (i, 0)),
            pl.BlockSpec((2, H1), lambda i: (0, 0)),
        ],
        out_shape=[
            jax.ShapeDtypeStruct((BL, H1), jnp.float32),
            jax.ShapeDtypeStruct((2, H1), jnp.float32),
        ],
    )(padded, tgt_emb, lenf, aW1, ab1.reshape(1, H1))

    H2 = aW2.shape[1]
    RB = 2048
    G2 = BL // RB
    h2, st2 = pl.pallas_call(
        functools.partial(_p2_body, float(BL)),
        grid=(G2,),
        in_specs=[
            pl.BlockSpec((RB, H1), lambda i: (i, 0)),
            pl.BlockSpec((2, H1), lambda i: (0, 0)),
            pl.BlockSpec((1, H1), lambda i: (0, 0)),
            pl.BlockSpec((H1, H2), lambda i: (0, 0)),
            pl.BlockSpec((1, H2), lambda i: (0, 0)),
        ],
        out_specs=[
            pl.BlockSpec((RB, H2), lambda i: (i, 0)),
            pl.BlockSpec((2, H2), lambda i: (0, 0)),
        ],
        out_shape=[
            jax.ShapeDtypeStruct((BL, H2), jnp.float32),
            jax.ShapeDtypeStruct((2, H2), jnp.float32),
        ],
    )(h1, st1, aa1.reshape(1, H1), aW2, ab2.reshape(1, H2))

    pooled = pl.pallas_call(
        functools.partial(_p3_body, float(BL), BB, LMAX, D),
        grid=(G1,),
        in_specs=[
            pl.BlockSpec((R, H2), lambda i: (i, 0)),
            pl.BlockSpec((2, H2), lambda i: (0, 0)),
            pl.BlockSpec((1, H2), lambda i: (0, 0)),
            pl.BlockSpec((1, H2), lambda i: (0, 0)),
            pl.BlockSpec((1, 1), lambda i: (0, 0)),
            pl.BlockSpec((R, D), lambda i: (i, 0)),
            pl.BlockSpec((BB, 1), lambda i: (i, 0)),
        ],
        out_specs=pl.BlockSpec((BB, D), lambda i: (i, 0)),
        out_shape=jax.ShapeDtypeStruct((B, D), jnp.float32),
    )(h2, st2, aa2.reshape(1, H2), aWo.reshape(1, H2), abo.reshape(1, 1),
      padded, lenf)

    wide_flat = wrows[:nf_flat].reshape(B, NF * WD)
    deep_flat = drows[:nf_flat].reshape(B, NF * deep_table.shape[1])
    M1 = mW1.shape[1]
    M2 = mW2.shape[1]
    DH1 = dW1.shape[1]
    DH2 = dW2.shape[1]
    full = lambda a, b: pl.BlockSpec((a, b), lambda: (0, 0))
    out = pl.pallas_call(
        functools.partial(_p4_body, B),
        in_specs=[
            full(B, D), full(B, D), full(B, D),
            full(3 * D, M1), full(1, M1), full(1, M1),
            full(M1, M2), full(1, M2), full(1, M2),
            full(1, M2), full(1, 1),
            full(B, NF * WD), full(1, NF * WD), full(1, 1),
            full(B, NF * deep_table.shape[1]),
            full(NF * deep_table.shape[1], DH1), full(1, DH1),
            full(DH1, DH2), full(1, DH2), full(1, DH2), full(1, 1),
        ],
        out_specs=full(B, 1),
        out_shape=jax.ShapeDtypeStruct((B, 1), jnp.float32),
        compiler_params=pltpu.CompilerParams(
            vmem_limit_bytes=100 * 1024 * 1024),
    )(oth_emb, pooled, tgt_emb,
      mW1, mb1.reshape(1, M1), ma1.reshape(1, M1),
      mW2, mb2.reshape(1, M2), ma2.reshape(1, M2),
      mWo.reshape(1, M2), mbo.reshape(1, 1),
      wide_flat, lrW.reshape(1, NF * WD), lrb.reshape(1, 1),
      deep_flat, dW1, db1.reshape(1, DH1),
      dW2, db2.reshape(1, DH2), dWo.reshape(1, DH2), dbo.reshape(1, 1))
    return out


# speedup vs baseline: 12.2486x; 1.1657x over previous
import functools

import jax
import jax.numpy as jnp
from jax import lax
from jax.experimental import pallas as pl
from jax.experimental.pallas import tpu as pltpu
from jax.experimental.pallas import tpu_sc as plsc

NC = 2
NS = 16
NW = NC * NS


def _sc_gather(addr, seq_flat_ids, din_table, deep_table, wide_table,
               deep_idx, wide_idx, oth_idx, tgt_idx,
               BL, D, WD, NFB, FTAIL):
    SEQ_CH = addr.shape[1]
    F_CH = deep_idx.shape[1]
    BPW = oth_idx.shape[1]
    B = NW * BPW
    SEQ_PW = SEQ_CH * 128
    F_PW = NFB // NW

    mesh = plsc.VectorSubcoreMesh(core_axis_name="c", subcore_axis_name="s")

    @functools.partial(
        pl.kernel,
        out_type=(
            jax.ShapeDtypeStruct((BL, D), jnp.float32),
            jax.ShapeDtypeStruct((NFB, D), jnp.float32),
            jax.ShapeDtypeStruct((NFB, WD), jnp.float32),
            jax.ShapeDtypeStruct((B, D), jnp.float32),
            jax.ShapeDtypeStruct((B, D), jnp.float32),
        ),
        mesh=mesh,
        compiler_params=pltpu.CompilerParams(use_tc_tiling_on_sc=False),
        scratch_types=[
            pltpu.VMEM((SEQ_CH, 128), jnp.int32),
            pltpu.VMEM((SEQ_CH, 128), jnp.int32),
            pltpu.VMEM((F_CH, 128), jnp.int32),
            pltpu.VMEM((2, BPW), jnp.int32),
            pltpu.VMEM((2, 128, D), jnp.float32),
            pltpu.VMEM((128, WD), jnp.float32),
            pltpu.VMEM((BPW, D), jnp.float32),
            pltpu.SemaphoreType.DMA,
            pltpu.SemaphoreType.DMA,
        ],
    )
    def k(addr_hbm, seqids_hbm, dinT_hbm, deepT_hbm, wideT_hbm,
          didx_hbm, widx_hbm, oidx_hbm, tidx_hbm,
          padded_hbm, drows_hbm, wrows_hbm, oemb_hbm, temb_hbm,
          addr_v, ids_v, fidx_v, smidx_v, rows_v, wrows_v, srows_v,
          semg, semw):
        c = lax.axis_index("c")
        s = lax.axis_index("s")
        wid = s * NC + c

        pltpu.sync_copy(addr_hbm.at[wid], addr_v)

        def g1(j, carry):
            pltpu.async_copy(seqids_hbm.at[addr_v.at[j]], ids_v.at[j], semg)
            return carry

        lax.fori_loop(0, SEQ_CH, g1, 0)

        def g1w(j, carry):
            pltpu.make_async_copy(seqids_hbm.at[addr_v.at[j]], ids_v.at[j],
                                  semg).wait()
            return carry

        lax.fori_loop(0, SEQ_CH, g1w, 0)

        sbase = wid * SEQ_PW
        pltpu.async_copy(dinT_hbm.at[ids_v.at[0]], rows_v.at[0], semg)

        def g2(j, carry):
            p = lax.rem(j, 2)
            pltpu.make_async_copy(dinT_hbm.at[ids_v.at[j]], rows_v.at[p],
                                  semg).wait()

            @pl.when(j >= 1)
            def _():
                pltpu.make_async_copy(
                    rows_v.at[1 - p],
                    padded_hbm.at[pl.ds(sbase, 128)], semw).wait()

            @pl.when(j + 1 < SEQ_CH)
            def _():
                pltpu.async_copy(dinT_hbm.at[ids_v.at[j + 1]],
                                 rows_v.at[1 - p], semg)

            pltpu.async_copy(rows_v.at[p],
                             padded_hbm.at[pl.ds(sbase + j * 128, 128)], semw)
            return carry

        lax.fori_loop(0, SEQ_CH, g2, 0)
        pltpu.make_async_copy(rows_v.at[0],
                              padded_hbm.at[pl.ds(sbase, 128)], semw).wait()

        pltpu.sync_copy(didx_hbm.at[wid], fidx_v)
        fbase = wid * F_PW

        def g3(j, carry):
            pltpu.async_copy(deepT_hbm.at[fidx_v.at[j]], rows_v.at[0],
                             semg).wait()
            pltpu.sync_copy(rows_v.at[0],
                            drows_hbm.at[pl.ds(fbase + j * 128, 128)])
            return carry

        lax.fori_loop(0, F_CH - 1, g3, 0)
        pltpu.async_copy(deepT_hbm.at[fidx_v.at[F_CH - 1]], rows_v.at[0],
                         semg).wait()
        pltpu.sync_copy(
            rows_v.at[0].at[pl.ds(0, FTAIL)],
            drows_hbm.at[pl.ds(fbase + (F_CH - 1) * 128, FTAIL)])

        pltpu.sync_copy(widx_hbm.at[wid], fidx_v)

        def g4(j, carry):
            pltpu.async_copy(wideT_hbm.at[fidx_v.at[j]], wrows_v,
                             semg).wait()
            pltpu.sync_copy(wrows_v,
                            wrows_hbm.at[pl.ds(fbase + j * 128, 128)])
            return carry

        lax.fori_loop(0, F_CH - 1, g4, 0)
        pltpu.async_copy(wideT_hbm.at[fidx_v.at[F_CH - 1]], wrows_v,
                         semg).wait()
        pltpu.sync_copy(
            wrows_v.at[pl.ds(0, FTAIL)],
            wrows_hbm.at[pl.ds(fbase + (F_CH - 1) * 128, FTAIL)])

        pltpu.sync_copy(oidx_hbm.at[wid], smidx_v.at[0])
        pltpu.sync_copy(tidx_hbm.at[wid], smidx_v.at[1])
        bbase = wid * BPW
        pltpu.async_copy(dinT_hbm.at[smidx_v.at[0]], srows_v, semg).wait()
        pltpu.sync_copy(srows_v, oemb_hbm.at[pl.ds(bbase, BPW)])
        pltpu.async_copy(dinT_hbm.at[smidx_v.at[1]], srows_v, semg).wait()
        pltpu.sync_copy(srows_v, temb_hbm.at[pl.ds(bbase, BPW)])

    return k(addr, seq_flat_ids, din_table, deep_table, wide_table,
             deep_idx, wide_idx, oth_idx, tgt_idx)


def _dice(x, st_ref, n, alpha_ref):
    st = st_ref[...]
    mean = st[0:1, :] * (1.0 / n)
    var = st[1:2, :] * (1.0 / n) - mean * mean
    xn = (x - mean) * lax.rsqrt(var + 1e-5)
    p = jax.nn.sigmoid(xn)
    return x * (p + (1.0 - p) * alpha_ref[...])


def _masked_p3(pad_ref, len_ref, BB, LMAX, D):
    p3 = pad_ref[...].reshape(BB, LMAX, D)
    liota = lax.broadcasted_iota(jnp.int32, (BB, LMAX, D), 1)
    liota = liota.astype(jnp.float32)
    len3 = len_ref[...].reshape(BB, 1, 1)
    return jnp.where(liota < len3, p3, 0.0)


def _p1_body(BB, LMAX, D, pad_ref, tgt_ref, len_ref, wA_ref, wB_ref, wC_ref,
             b_ref, h_ref, st_ref):
    i = pl.program_id(0)
    R = BB * LMAX
    p3 = _masked_p3(pad_ref, len_ref, BB, LMAX, D)
    t3 = jnp.broadcast_to(tgt_ref[...].reshape(BB, 1, D), (BB, LMAX, D))
    pb = p3.reshape(R, D).astype(jnp.bfloat16)
    tpb = (t3 * p3).reshape(R, D).astype(jnp.bfloat16)
    h = jnp.dot(pb, wB_ref[...], preferred_element_type=jnp.float32)
    h += jnp.dot(tpb, wC_ref[...], preferred_element_type=jnp.float32)
    ta = jnp.dot(tgt_ref[...].astype(jnp.bfloat16), wA_ref[...],
                 preferred_element_type=jnp.float32)
    H = ta.shape[1]
    h = (h.reshape(BB, LMAX, H) + ta.reshape(BB, 1, H)).reshape(R, H)
    h = h + b_ref[...]
    h_ref[...] = h.astype(jnp.bfloat16)

    @pl.when(i == 0)
    def _():
        st_ref[...] = jnp.zeros_like(st_ref)

    st_ref[0:1, :] += jnp.sum(h, axis=0, keepdims=True)
    st_ref[1:2, :] += jnp.sum(h * h, axis=0, keepdims=True)


def _p2_body(n1, h1_ref, st1_ref, a1_ref, w_ref, b_ref, h2_ref, st2_ref):
    i = pl.program_id(0)
    x = _dice(h1_ref[...].astype(jnp.float32), st1_ref, n1, a1_ref)
    h2 = jnp.dot(x.astype(jnp.bfloat16), w_ref[...],
                 preferred_element_type=jnp.float32)
    h2 = h2 + b_ref[...]
    h2_ref[...] = h2.astype(jnp.bfloat16)

    @pl.when(i == 0)
    def _():
        st2_ref[...] = jnp.zeros_like(st2_ref)

    st2_ref[0:1, :] += jnp.sum(h2, axis=0, keepdims=True)
    st2_ref[1:2, :] += jnp.sum(h2 * h2, axis=0, keepdims=True)


def _p3_body(n1, BB, LMAX, D, h2_ref, st2_ref, a2_ref, wo_ref, bo_ref,
             pad_ref, len_ref, pooled_ref):
    x = _dice(h2_ref[...].astype(jnp.float32), st2_ref, n1, a2_ref)
    s = jnp.sum(x * wo_ref[...], axis=1, keepdims=True) + bo_ref[...]
    s3 = s.reshape(BB, LMAX, 1)
    liota = lax.broadcasted_iota(jnp.int32, (BB, LMAX, 1), 1)
    liota = liota.astype(jnp.float32)
    len3 = len_ref[...].reshape(BB, 1, 1)
    s3 = jnp.where(liota < len3, s3, 0.0)
    p3 = pad_ref[...].reshape(BB, LMAX, D)
    pooled_ref[...] = jnp.sum(p3 * s3, axis=1)


def _bn_in(x, n):
    mean = jnp.sum(x, axis=0, keepdims=True) * (1.0 / n)
    d = x - mean
    var = jnp.sum(d * d, axis=0, keepdims=True) * (1.0 / n)
    return d * lax.rsqrt(var + 1e-5)


def _dice_in(x, n, alpha):
    xn = _bn_in(x, n)
    p = jax.nn.sigmoid(xn)
    return x * (p + (1.0 - p) * alpha)


def _p4_body(B,
             oth_ref, pool_ref, tgt_ref,
             mW1_ref, mb1_ref, ma1_ref, mW2_ref, mb2_ref, ma2_ref,
             mWo_ref, mbo_ref,
             wide_ref, lrW_ref, lrb_ref,
             deep_ref, dW1_ref, db1_ref, dW2_ref, db2_ref, dWo_ref, dbo_ref,
             out_ref):
    n = float(B)
    bf = jnp.bfloat16
    emb = jnp.concatenate([oth_ref[...], pool_ref[...], tgt_ref[...]], axis=1)
    x = jnp.dot(emb.astype(bf), mW1_ref[...], preferred_element_type=jnp.float32)
    x = _dice_in(x + mb1_ref[...], n, ma1_ref[...])
    x = jnp.dot(x.astype(bf), mW2_ref[...], preferred_element_type=jnp.float32)
    x = _dice_in(x + mb2_ref[...], n, ma2_ref[...])
    din = jnp.sum(x * mWo_ref[...], axis=1, keepdims=True) + mbo_ref[...]

    lr = jnp.sum(wide_ref[...] * lrW_ref[...], axis=1, keepdims=True)
    lr = lr + lrb_ref[...]

    d = jnp.dot(deep_ref[...].astype(bf), dW1_ref[...],
                preferred_element_type=jnp.float32)
    d = jax.nn.relu(_bn_in(d + db1_ref[...], n))
    d = jnp.dot(d.astype(bf), dW2_ref[...], preferred_element_type=jnp.float32)
    d = jax.nn.relu(_bn_in(d + db2_ref[...], n))
    deep = jnp.sum(d * dWo_ref[...], axis=1, keepdims=True) + dbo_ref[...]

    out_ref[...] = jax.nn.sigmoid(din + lr + deep)


def kernel(other_ids, seq_flat_ids, cu_seqlens, target_ids, wide_ids, deep_ids,
           din_table, aW1, ab1, aa1, aW2, ab2, aa2, aWo, abo,
           mW1, mb1, ma1, mW2, mb2, ma2, mWo, mbo,
           wide_table, lrW, lrb, deep_table, dW1, db1, dW2, db2, dWo, dbo):
    B = other_ids.shape[0]
    T = seq_flat_ids.shape[0]
    D = din_table.shape[1]
    WD = wide_table.shape[1]
    DD = deep_table.shape[1]
    NF = wide_ids.shape[1]
    LMAX = 200
    BL = B * LMAX

    cu = cu_seqlens.astype(jnp.int32)
    lengths = cu[1:] - cu[:-1]
    addr = cu[:-1, None] + jnp.arange(LMAX, dtype=jnp.int32)[None, :]
    addr = jnp.minimum(addr, T - 1).reshape(NW, BL // NW // 128, 128)

    nf_flat = B * NF
    F_PW = nf_flat // NW
    F_CH = -(-F_PW // 128)
    FTAIL = F_PW - (F_CH - 1) * 128
    zpad = jnp.zeros((NW, F_CH * 128 - F_PW), jnp.int32)
    didx = jnp.concatenate(
        [deep_ids.reshape(NW, F_PW).astype(jnp.int32), zpad], axis=1)
    widx = jnp.concatenate(
        [wide_ids.reshape(NW, F_PW).astype(jnp.int32), zpad], axis=1)
    didx = didx.reshape(NW, F_CH, 128)
    widx = widx.reshape(NW, F_CH, 128)
    oidx = other_ids.astype(jnp.int32).reshape(NW, B // NW)
    tidx = target_ids.astype(jnp.int32).reshape(NW, B // NW)

    padded, drows, wrows, oth_emb, tgt_emb = _sc_gather(
        addr, seq_flat_ids.astype(jnp.int32), din_table, deep_table,
        wide_table, didx, widx, oidx, tidx, BL, D, WD, nf_flat, FTAIL)

    lenf = lengths.astype(jnp.float32).reshape(B, 1)

    bf = jnp.bfloat16
    wA = (aW1[:D] + aW1[2 * D:3 * D]).astype(bf)
    wB = (aW1[D:2 * D] - aW1[2 * D:3 * D]).astype(bf)
    wC = aW1[3 * D:].astype(bf)

    BB = 8
    G1 = B // BB
    R = BB * LMAX
    H1 = aW1.shape[1]
    h1, st1 = pl.pallas_call(
        functools.partial(_p1_body, BB, LMAX, D),
        grid=(G1,),
        in_specs=[
            pl.BlockSpec((R, D), lambda i: (i, 0)),
            pl.BlockSpec((BB, D), lambda i: (i, 0)),
            pl.BlockSpec((BB, 1), lambda i: (i, 0)),
            pl.BlockSpec((D, H1), lambda i: (0, 0)),
            pl.BlockSpec((D, H1), lambda i: (0, 0)),
            pl.BlockSpec((D, H1), lambda i: (0, 0)),
            pl.BlockSpec((1, H1), lambda i: (0, 0)),
        ],
        out_specs=[
            pl.BlockSpec((R, H1), lambda i: (i, 0)),
            pl.BlockSpec((2, H1), lambda i: (0, 0)),
        ],
        out_shape=[
            jax.ShapeDtypeStruct((BL, H1), bf),
            jax.ShapeDtypeStruct((2, H1), jnp.float32),
        ],
    )(padded, tgt_emb, lenf, wA, wB, wC, ab1.reshape(1, H1))

    H2 = aW2.shape[1]
    RB = 2048
    G2 = BL // RB
    h2, st2 = pl.pallas_call(
        functools.partial(_p2_body, float(BL)),
        grid=(G2,),
        in_specs=[
            pl.BlockSpec((RB, H1), lambda i: (i, 0)),
            pl.BlockSpec((2, H1), lambda i: (0, 0)),
            pl.BlockSpec((1, H1), lambda i: (0, 0)),
            pl.BlockSpec((H1, H2), lambda i: (0, 0)),
            pl.BlockSpec((1, H2), lambda i: (0, 0)),
        ],
        out_specs=[
            pl.BlockSpec((RB, H2), lambda i: (i, 0)),
            pl.BlockSpec((2, H2), lambda i: (0, 0)),
        ],
        out_shape=[
            jax.ShapeDtypeStruct((BL, H2), bf),
            jax.ShapeDtypeStruct((2, H2), jnp.float32),
        ],
    )(h1, st1, aa1.reshape(1, H1), aW2.astype(bf), ab2.reshape(1, H2))

    pooled = pl.pallas_call(
        functools.partial(_p3_body, float(BL), BB, LMAX, D),
        grid=(G1,),
        in_specs=[
            pl.BlockSpec((R, H2), lambda i: (i, 0)),
            pl.BlockSpec((2, H2), lambda i: (0, 0)),
            pl.BlockSpec((1, H2), lambda i: (0, 0)),
            pl.BlockSpec((1, H2), lambda i: (0, 0)),
            pl.BlockSpec((1, 1), lambda i: (0, 0)),
            pl.BlockSpec((R, D), lambda i: (i, 0)),
            pl.BlockSpec((BB, 1), lambda i: (i, 0)),
        ],
        out_specs=pl.BlockSpec((BB, D), lambda i: (i, 0)),
        out_shape=jax.ShapeDtypeStruct((B, D), jnp.float32),
    )(h2, st2, aa2.reshape(1, H2), aWo.reshape(1, H2), abo.reshape(1, 1),
      padded, lenf)

    wide_flat = wrows.reshape(B, NF * WD)
    deep_flat = drows.reshape(B, NF * DD)
    M1 = mW1.shape[1]
    M2 = mW2.shape[1]
    DH1 = dW1.shape[1]
    DH2 = dW2.shape[1]
    full = lambda a, b: pl.BlockSpec((a, b), lambda: (0, 0))
    out = pl.pallas_call(
        functools.partial(_p4_body, B),
        in_specs=[
            full(B, D), full(B, D), full(B, D),
            full(3 * D, M1), full(1, M1), full(1, M1),
            full(M1, M2), full(1, M2), full(1, M2),
            full(1, M2), full(1, 1),
            full(B, NF * WD), full(1, NF * WD), full(1, 1),
            full(B, NF * DD),
            full(NF * DD, DH1), full(1, DH1),
            full(DH1, DH2), full(1, DH2), full(1, DH2), full(1, 1),
        ],
        out_specs=full(B, 1),
        out_shape=jax.ShapeDtypeStruct((B, 1), jnp.float32),
        compiler_params=pltpu.CompilerParams(
            vmem_limit_bytes=100 * 1024 * 1024),
    )(oth_emb, pooled, tgt_emb,
      mW1.astype(bf), mb1.reshape(1, M1), ma1.reshape(1, M1),
      mW2.astype(bf), mb2.reshape(1, M2), ma2.reshape(1, M2),
      mWo.reshape(1, M2), mbo.reshape(1, 1),
      wide_flat, lrW.reshape(1, NF * WD), lrb.reshape(1, 1),
      deep_flat, dW1.astype(bf), db1.reshape(1, DH1),
      dW2.astype(bf), db2.reshape(1, DH2), dWo.reshape(1, DH2),
      dbo.reshape(1, 1))
    return out


# speedup vs baseline: 12.7718x; 1.0427x over previous
import functools

import jax
import jax.numpy as jnp
from jax import lax
from jax.experimental import pallas as pl
from jax.experimental.pallas import tpu as pltpu
from jax.experimental.pallas import tpu_sc as plsc

NC = 2
NS = 16
NW = NC * NS
NBUF = 4


def _sc_seq_gather(addr, seq_flat_ids, din_table, oth_idx, tgt_idx, BL, D):
    SEQ_CH = addr.shape[1]
    BPW = oth_idx.shape[1]
    B = NW * BPW
    SEQ_PW = SEQ_CH * 128

    mesh = plsc.VectorSubcoreMesh(core_axis_name="c", subcore_axis_name="s")

    @functools.partial(
        pl.kernel,
        out_type=(
            jax.ShapeDtypeStruct((BL, D), jnp.float32),
            jax.ShapeDtypeStruct((B, D), jnp.float32),
            jax.ShapeDtypeStruct((B, D), jnp.float32),
        ),
        mesh=mesh,
        compiler_params=pltpu.CompilerParams(use_tc_tiling_on_sc=False),
        scratch_types=[
            pltpu.VMEM((SEQ_CH, 128), jnp.int32),
            pltpu.VMEM((SEQ_CH, 128), jnp.int32),
            pltpu.VMEM((2, BPW), jnp.int32),
            pltpu.VMEM((NBUF, 128, D), jnp.float32),
            pltpu.VMEM((BPW, D), jnp.float32),
        ] + [pltpu.SemaphoreType.DMA] * (2 * NBUF + 1),
    )
    def k(addr_hbm, seqids_hbm, dinT_hbm, oidx_hbm, tidx_hbm,
          padded_hbm, oemb_hbm, temb_hbm,
          addr_v, ids_v, smidx_v, rows_v, srows_v, *sems):
        semg = sems[:NBUF]
        semw = sems[NBUF:2 * NBUF]
        sem1 = sems[2 * NBUF]
        c = lax.axis_index("c")
        s = lax.axis_index("s")
        wid = s * NC + c

        pltpu.sync_copy(addr_hbm.at[wid], addr_v)

        def g1(j, carry):
            pltpu.async_copy(seqids_hbm.at[addr_v.at[j]], ids_v.at[j], sem1)
            return carry

        lax.fori_loop(0, SEQ_CH, g1, 0)

        def g1w(j, carry):
            pltpu.make_async_copy(seqids_hbm.at[addr_v.at[j]], ids_v.at[j],
                                  sem1).wait()
            return carry

        lax.fori_loop(0, SEQ_CH, g1w, 0)

        sbase = wid * SEQ_PW

        def wb_dst(j):
            return padded_hbm.at[pl.ds(sbase + j * 128, 128)]

        for p in range(NBUF - 1):
            pltpu.async_copy(dinT_hbm.at[ids_v.at[p]], rows_v.at[p], semg[p])

        def g2(j, carry):
            for pp in range(NBUF):
                @pl.when(lax.rem(j, NBUF) == pp)
                def _(pp=pp):
                    qq = (pp + NBUF - 1) % NBUF
                    pltpu.make_async_copy(dinT_hbm.at[ids_v.at[j]],
                                          rows_v.at[pp], semg[pp]).wait()

                    @pl.when(j + NBUF - 1 < SEQ_CH)
                    def _():
                        @pl.when(j >= 1)
                        def _():
                            pltpu.make_async_copy(rows_v.at[qq],
                                                  wb_dst(j - 1),
                                                  semw[qq]).wait()

                        pltpu.async_copy(dinT_hbm.at[ids_v.at[j + NBUF - 1]],
                                         rows_v.at[qq], semg[qq])

                    pltpu.async_copy(rows_v.at[pp], wb_dst(j), semw[pp])
            return carry

        lax.fori_loop(0, SEQ_CH, g2, 0)
        for pp in range(NBUF):
            pltpu.make_async_copy(rows_v.at[pp], wb_dst(0), semw[pp]).wait()

        pltpu.sync_copy(oidx_hbm.at[wid], smidx_v.at[0])
        pltpu.sync_copy(tidx_hbm.at[wid], smidx_v.at[1])
        bbase = wid * BPW
        pltpu.async_copy(dinT_hbm.at[smidx_v.at[0]], srows_v, sem1).wait()
        pltpu.sync_copy(srows_v, oemb_hbm.at[pl.ds(bbase, BPW)])
        pltpu.async_copy(dinT_hbm.at[smidx_v.at[1]], srows_v, sem1).wait()
        pltpu.sync_copy(srows_v, temb_hbm.at[pl.ds(bbase, BPW)])

    return k(addr, seq_flat_ids, din_table, oth_idx, tgt_idx)


def _sc_feat_gather(deep_table, wide_table, deep_idx, wide_idx,
                    NFB, DD, WD, FTAIL):
    F_CH = deep_idx.shape[1]
    F_PW = NFB // NW

    mesh = plsc.VectorSubcoreMesh(core_axis_name="c", subcore_axis_name="s")

    @functools.partial(
        pl.kernel,
        out_type=(
            jax.ShapeDtypeStruct((NFB, DD), jnp.float32),
            jax.ShapeDtypeStruct((NFB, WD), jnp.float32),
        ),
        mesh=mesh,
        compiler_params=pltpu.CompilerParams(use_tc_tiling_on_sc=False),
        scratch_types=[
            pltpu.VMEM((F_CH, 128), jnp.int32),
            pltpu.VMEM((2, 128, DD), jnp.float32),
            pltpu.VMEM((128, WD), jnp.float32),
            pltpu.SemaphoreType.DMA,
            pltpu.SemaphoreType.DMA,
            pltpu.SemaphoreType.DMA,
        ],
    )
    def k(deepT_hbm, wideT_hbm, didx_hbm, widx_hbm,
          drows_hbm, wrows_hbm,
          fidx_v, rows_v, wrows_v, semg, semw, sem1):
        c = lax.axis_index("c")
        s = lax.axis_index("s")
        wid = s * NC + c
        fbase = wid * F_PW

        pltpu.sync_copy(didx_hbm.at[wid], fidx_v)
        pltpu.async_copy(deepT_hbm.at[fidx_v.at[0]], rows_v.at[0], semg)

        def g3(j, carry):
            for pp in range(2):
                @pl.when(lax.rem(j, 2) == pp)
                def _(pp=pp):
                    pltpu.make_async_copy(deepT_hbm.at[fidx_v.at[j]],
                                          rows_v.at[pp], semg).wait()

                    @pl.when(j + 1 < F_CH)
                    def _():
                        @pl.when(j >= 1)
                        def _():
                            pltpu.make_async_copy(
                                rows_v.at[1 - pp],
                                drows_hbm.at[pl.ds(fbase, 128)], semw).wait()

                        pltpu.async_copy(deepT_hbm.at[fidx_v.at[j + 1]],
                                         rows_v.at[1 - pp], semg)

                    @pl.when(j < F_CH - 1)
                    def _():
                        pltpu.async_copy(
                            rows_v.at[pp],
                            drows_hbm.at[pl.ds(fbase + j * 128, 128)], semw)

                    @pl.when(j == F_CH - 1)
                    def _():
                        pltpu.async_copy(
                            rows_v.at[pp].at[pl.ds(0, FTAIL)],
                            drows_hbm.at[pl.ds(fbase + j * 128, FTAIL)],
                            semw)

            return carry

        lax.fori_loop(0, F_CH, g3, 0)
        pltpu.make_async_copy(rows_v.at[0],
                              drows_hbm.at[pl.ds(fbase, 128)], semw).wait()
        pltpu.make_async_copy(rows_v.at[0].at[pl.ds(0, FTAIL)],
                              drows_hbm.at[pl.ds(fbase, FTAIL)], semw).wait()

        pltpu.sync_copy(widx_hbm.at[wid], fidx_v)

        def g4(j, carry):
            pltpu.async_copy(wideT_hbm.at[fidx_v.at[j]], wrows_v,
                             sem1).wait()
            pltpu.sync_copy(wrows_v,
                            wrows_hbm.at[pl.ds(fbase + j * 128, 128)])
            return carry

        lax.fori_loop(0, F_CH - 1, g4, 0)
        pltpu.async_copy(wideT_hbm.at[fidx_v.at[F_CH - 1]], wrows_v,
                         sem1).wait()
        pltpu.sync_copy(
            wrows_v.at[pl.ds(0, FTAIL)],
            wrows_hbm.at[pl.ds(fbase + (F_CH - 1) * 128, FTAIL)])

    return k(deep_table, wide_table, deep_idx, wide_idx)


def _dice(x, st_ref, n, alpha_ref):
    st = st_ref[...]
    mean = st[0:1, :] * (1.0 / n)
    var = st[1:2, :] * (1.0 / n) - mean * mean
    xn = (x - mean) * lax.rsqrt(var + 1e-5)
    p = jax.nn.sigmoid(xn)
    return x * (p + (1.0 - p) * alpha_ref[...])


def _masked_p3(pad_ref, len_ref, BB, LMAX, D):
    p3 = pad_ref[...].reshape(BB, LMAX, D)
    liota = lax.broadcasted_iota(jnp.int32, (BB, LMAX, D), 1)
    liota = liota.astype(jnp.float32)
    len3 = len_ref[...].reshape(BB, 1, 1)
    return jnp.where(liota < len3, p3, 0.0)


def _p1_body(BB, LMAX, D, pad_ref, tgt_ref, len_ref, wA_ref, wB_ref, wC_ref,
             b_ref, h_ref, st_ref):
    i = pl.program_id(0)
    R = BB * LMAX
    p3 = _masked_p3(pad_ref, len_ref, BB, LMAX, D)
    t3 = jnp.broadcast_to(tgt_ref[...].reshape(BB, 1, D), (BB, LMAX, D))
    pb = p3.reshape(R, D).astype(jnp.bfloat16)
    tpb = (t3 * p3).reshape(R, D).astype(jnp.bfloat16)
    h = jnp.dot(pb, wB_ref[...], preferred_element_type=jnp.float32)
    h += jnp.dot(tpb, wC_ref[...], preferred_element_type=jnp.float32)
    ta = jnp.dot(tgt_ref[...].astype(jnp.bfloat16), wA_ref[...],
                 preferred_element_type=jnp.float32)
    H = ta.shape[1]
    h = (h.reshape(BB, LMAX, H) + ta.reshape(BB, 1, H)).reshape(R, H)
    h = h + b_ref[...]
    h_ref[...] = h.astype(jnp.bfloat16)

    @pl.when(i == 0)
    def _():
        st_ref[...] = jnp.zeros_like(st_ref)

    st_ref[0:1, :] += jnp.sum(h, axis=0, keepdims=True)
    st_ref[1:2, :] += jnp.sum(h * h, axis=0, keepdims=True)


def _p2_body(n1, h1_ref, st1_ref, a1_ref, w_ref, b_ref, h2_ref, st2_ref):
    i = pl.program_id(0)
    x = _dice(h1_ref[...].astype(jnp.float32), st1_ref, n1, a1_ref)
    h2 = jnp.dot(x.astype(jnp.bfloat16), w_ref[...],
                 preferred_element_type=jnp.float32)
    h2 = h2 + b_ref[...]
    h2_ref[...] = h2.astype(jnp.bfloat16)

    @pl.when(i == 0)
    def _():
        st2_ref[...] = jnp.zeros_like(st2_ref)

    st2_ref[0:1, :] += jnp.sum(h2, axis=0, keepdims=True)
    st2_ref[1:2, :] += jnp.sum(h2 * h2, axis=0, keepdims=True)


def _bn_in(x, n):
    mean = jnp.sum(x, axis=0, keepdims=True) * (1.0 / n)
    d = x - mean
    var = jnp.sum(d * d, axis=0, keepdims=True) * (1.0 / n)
    return d * lax.rsqrt(var + 1e-5)


def _dice_in(x, n, alpha):
    xn = _bn_in(x, n)
    p = jax.nn.sigmoid(xn)
    return x * (p + (1.0 - p) * alpha)


def _p34_body(n1, BB, LMAX, D, G1, B,
              h2_ref, st2_ref, a2_ref, wo_ref, bo_ref, pad_ref, len_ref,
              oth_ref, tgt_ref,
              mW1_ref, mb1_ref, ma1_ref, mW2_ref, mb2_ref, ma2_ref,
              mWo_ref, mbo_ref,
              wide_ref, lrW_ref, lrb_ref,
              deep_ref, dW1_ref, db1_ref, dW2_ref, db2_ref, dWo_ref, dbo_ref,
              out_ref, pool_v):
    i = pl.program_id(0)

    @pl.when(i < G1)
    def _():
        x = _dice(h2_ref[...].astype(jnp.float32), st2_ref, n1, a2_ref)
        s = jnp.sum(x * wo_ref[...], axis=1, keepdims=True) + bo_ref[...]
        s3 = s.reshape(BB, LMAX, 1)
        liota = lax.broadcasted_iota(jnp.int32, (BB, LMAX, 1), 1)
        liota = liota.astype(jnp.float32)
        len3 = len_ref[...].reshape(BB, 1, 1)
        s3 = jnp.where(liota < len3, s3, 0.0)
        p3 = pad_ref[...].reshape(BB, LMAX, D)
        pool_v[pl.ds(i * BB, BB), :] = jnp.sum(p3 * s3, axis=1)

    @pl.when(i == G1)
    def _():
        n = float(B)
        bf = jnp.bfloat16
        emb = jnp.concatenate(
            [oth_ref[...], pool_v[...], tgt_ref[...]], axis=1)
        x = jnp.dot(emb.astype(bf), mW1_ref[...],
                    preferred_element_type=jnp.float32)
        x = _dice_in(x + mb1_ref[...], n, ma1_ref[...])
        x = jnp.dot(x.astype(bf), mW2_ref[...],
                    preferred_element_type=jnp.float32)
        x = _dice_in(x + mb2_ref[...], n, ma2_ref[...])
        din = jnp.sum(x * mWo_ref[...], axis=1, keepdims=True) + mbo_ref[...]

        lr = jnp.sum(wide_ref[...] * lrW_ref[...], axis=1, keepdims=True)
        lr = lr + lrb_ref[...]

        d = jnp.dot(deep_ref[...].astype(bf), dW1_ref[...],
                    preferred_element_type=jnp.float32)
        d = jax.nn.relu(_bn_in(d + db1_ref[...], n))
        d = jnp.dot(d.astype(bf), dW2_ref[...],
                    preferred_element_type=jnp.float32)
        d = jax.nn.relu(_bn_in(d + db2_ref[...], n))
        deep = jnp.sum(d * dWo_ref[...], axis=1, keepdims=True) + dbo_ref[...]

        out_ref[...] = jax.nn.sigmoid(din + lr + deep)


def kernel(other_ids, seq_flat_ids, cu_seqlens, target_ids, wide_ids, deep_ids,
           din_table, aW1, ab1, aa1, aW2, ab2, aa2, aWo, abo,
           mW1, mb1, ma1, mW2, mb2, ma2, mWo, mbo,
           wide_table, lrW, lrb, deep_table, dW1, db1, dW2, db2, dWo, dbo):
    B = other_ids.shape[0]
    T = seq_flat_ids.shape[0]
    D = din_table.shape[1]
    WD = wide_table.shape[1]
    DD = deep_table.shape[1]
    NF = wide_ids.shape[1]
    LMAX = 200
    BL = B * LMAX

    cu = cu_seqlens.astype(jnp.int32)
    lengths = cu[1:] - cu[:-1]
    addr = cu[:-1, None] + jnp.arange(LMAX, dtype=jnp.int32)[None, :]
    addr = jnp.minimum(addr, T - 1).reshape(NW, BL // NW // 128, 128)

    nf_flat = B * NF
    F_PW = nf_flat // NW
    F_CH = -(-F_PW // 128)
    FTAIL = F_PW - (F_CH - 1) * 128
    zpad = jnp.zeros((NW, F_CH * 128 - F_PW), jnp.int32)
    didx = jnp.concatenate(
        [deep_ids.reshape(NW, F_PW).astype(jnp.int32), zpad], axis=1)
    widx = jnp.concatenate(
        [wide_ids.reshape(NW, F_PW).astype(jnp.int32), zpad], axis=1)
    didx = didx.reshape(NW, F_CH, 128)
    widx = widx.reshape(NW, F_CH, 128)
    oidx = other_ids.astype(jnp.int32).reshape(NW, B // NW)
    tidx = target_ids.astype(jnp.int32).reshape(NW, B // NW)

    padded, oth_emb, tgt_emb = _sc_seq_gather(
        addr, seq_flat_ids.astype(jnp.int32), din_table, oidx, tidx, BL, D)
    drows, wrows = _sc_feat_gather(
        deep_table, wide_table, didx, widx, nf_flat, DD, WD, FTAIL)

    lenf = lengths.astype(jnp.float32).reshape(B, 1)

    bf = jnp.bfloat16
    wA = (aW1[:D] + aW1[2 * D:3 * D]).astype(bf)
    wB = (aW1[D:2 * D] - aW1[2 * D:3 * D]).astype(bf)
    wC = aW1[3 * D:].astype(bf)

    BB = 8
    G1 = B // BB
    R = BB * LMAX
    H1 = aW1.shape[1]
    h1, st1 = pl.pallas_call(
        functools.partial(_p1_body, BB, LMAX, D),
        grid=(G1,),
        in_specs=[
            pl.BlockSpec((R, D), lambda i: (i, 0)),
            pl.BlockSpec((BB, D), lambda i: (i, 0)),
            pl.BlockSpec((BB, 1), lambda i: (i, 0)),
            pl.BlockSpec((D, H1), lambda i: (0, 0)),
            pl.BlockSpec((D, H1), lambda i: (0, 0)),
            pl.BlockSpec((D, H1), lambda i: (0, 0)),
            pl.BlockSpec((1, H1), lambda i: (0, 0)),
        ],
        out_specs=[
            pl.BlockSpec((R, H1), lambda i: (i, 0)),
            pl.BlockSpec((2, H1), lambda i: (0, 0)),
        ],
        out_shape=[
            jax.ShapeDtypeStruct((BL, H1), bf),
            jax.ShapeDtypeStruct((2, H1), jnp.float32),
        ],
    )(padded, tgt_emb, lenf, wA, wB, wC, ab1.reshape(1, H1))

    H2 = aW2.shape[1]
    RB = 2048
    G2 = BL // RB
    h2, st2 = pl.pallas_call(
        functools.partial(_p2_body, float(BL)),
        grid=(G2,),
        in_specs=[
            pl.BlockSpec((RB, H1), lambda i: (i, 0)),
            pl.BlockSpec((2, H1), lambda i: (0, 0)),
            pl.BlockSpec((1, H1), lambda i: (0, 0)),
            pl.BlockSpec((H1, H2), lambda i: (0, 0)),
            pl.BlockSpec((1, H2), lambda i: (0, 0)),
        ],
        out_specs=[
            pl.BlockSpec((RB, H2), lambda i: (i, 0)),
            pl.BlockSpec((2, H2), lambda i: (0, 0)),
        ],
        out_shape=[
            jax.ShapeDtypeStruct((BL, H2), bf),
            jax.ShapeDtypeStruct((2, H2), jnp.float32),
        ],
    )(h1, st1, aa1.reshape(1, H1), aW2.astype(bf), ab2.reshape(1, H2))

    wide_flat = wrows.reshape(B, NF * WD)
    deep_flat = drows.reshape(B, NF * DD)
    M1 = mW1.shape[1]
    M2 = mW2.shape[1]
    DH1 = dW1.shape[1]
    DH2 = dW2.shape[1]
    li = lambda i: (jnp.minimum(i, G1 - 1), 0)
    cst = lambda i: (0, 0)
    out = pl.pallas_call(
        functools.partial(_p34_body, float(BL), BB, LMAX, D, G1, B),
        grid=(G1 + 1,),
        in_specs=[
            pl.BlockSpec((R, H2), li),
            pl.BlockSpec((2, H2), cst),
            pl.BlockSpec((1, H2), cst),
            pl.BlockSpec((1, H2), cst),
            pl.BlockSpec((1, 1), cst),
            pl.BlockSpec((R, D), li),
            pl.BlockSpec((BB, 1), li),
            pl.BlockSpec((B, D), cst),
            pl.BlockSpec((B, D), cst),
            pl.BlockSpec((3 * D, M1), cst),
            pl.BlockSpec((1, M1), cst),
            pl.BlockSpec((1, M1), cst),
            pl.BlockSpec((M1, M2), cst),
            pl.BlockSpec((1, M2), cst),
            pl.BlockSpec((1, M2), cst),
            pl.BlockSpec((1, M2), cst),
            pl.BlockSpec((1, 1), cst),
            pl.BlockSpec((B, NF * WD), cst),
            pl.BlockSpec((1, NF * WD), cst),
            pl.BlockSpec((1, 1), cst),
            pl.BlockSpec((B, NF * DD), cst),
            pl.BlockSpec((NF * DD, DH1), cst),
            pl.BlockSpec((1, DH1), cst),
            pl.BlockSpec((DH1, DH2), cst),
            pl.BlockSpec((1, DH2), cst),
            pl.BlockSpec((1, DH2), cst),
            pl.BlockSpec((1, 1), cst),
        ],
        out_specs=pl.BlockSpec((B, 1), cst),
        out_shape=jax.ShapeDtypeStruct((B, 1), jnp.float32),
        scratch_shapes=[pltpu.VMEM((B, D), jnp.float32)],
        compiler_params=pltpu.CompilerParams(
            vmem_limit_bytes=100 * 1024 * 1024),
    )(h2, st2, aa2.reshape(1, H2), aWo.reshape(1, H2), abo.reshape(1, 1),
      padded, lenf,
      oth_emb, tgt_emb,
      mW1.astype(bf), mb1.reshape(1, M1), ma1.reshape(1, M1),
      mW2.astype(bf), mb2.reshape(1, M2), ma2.reshape(1, M2),
      mWo.reshape(1, M2), mbo.reshape(1, 1),
      wide_flat, lrW.reshape(1, NF * WD), lrb.reshape(1, 1),
      deep_flat, dW1.astype(bf), db1.reshape(1, DH1),
      dW2.astype(bf), db2.reshape(1, DH2), dWo.reshape(1, DH2),
      dbo.reshape(1, 1))
    return out


# speedup vs baseline: 13.0218x; 1.0196x over previous
import functools

import jax
import jax.numpy as jnp
from jax import lax
from jax.experimental import pallas as pl
from jax.experimental.pallas import tpu as pltpu
from jax.experimental.pallas import tpu_sc as plsc

NC = 2
NS = 16
NW = NC * NS
NBUF = 4


def _sc_seq_gather(addr, seq_flat_ids, din_table, oth_idx, tgt_idx, BL, D):
    SEQ_CH = addr.shape[1]
    BPW = oth_idx.shape[1]
    B = NW * BPW
    SEQ_PW = SEQ_CH * 128

    mesh = plsc.VectorSubcoreMesh(core_axis_name="c", subcore_axis_name="s")

    @functools.partial(
        pl.kernel,
        out_type=(
            jax.ShapeDtypeStruct((BL, D), jnp.float32),
            jax.ShapeDtypeStruct((B, D), jnp.float32),
            jax.ShapeDtypeStruct((B, D), jnp.float32),
        ),
        mesh=mesh,
        compiler_params=pltpu.CompilerParams(use_tc_tiling_on_sc=False),
        scratch_types=[
            pltpu.VMEM((SEQ_CH, 128), jnp.int32),
            pltpu.VMEM((SEQ_CH, 128), jnp.int32),
            pltpu.VMEM((2, BPW), jnp.int32),
            pltpu.VMEM((NBUF, 128, D), jnp.float32),
            pltpu.VMEM((BPW, D), jnp.float32),
        ] + [pltpu.SemaphoreType.DMA] * (2 * NBUF + 1),
    )
    def k(addr_hbm, seqids_hbm, dinT_hbm, oidx_hbm, tidx_hbm,
          padded_hbm, oemb_hbm, temb_hbm,
          addr_v, ids_v, smidx_v, rows_v, srows_v, *sems):
        semg = sems[:NBUF]
        semw = sems[NBUF:2 * NBUF]
        sem1 = sems[2 * NBUF]
        c = lax.axis_index("c")
        s = lax.axis_index("s")
        wid = s * NC + c

        pltpu.sync_copy(addr_hbm.at[wid], addr_v)

        def g1(j, carry):
            pltpu.async_copy(seqids_hbm.at[addr_v.at[j]], ids_v.at[j], sem1)
            return carry

        lax.fori_loop(0, SEQ_CH, g1, 0)

        def g1w(j, carry):
            pltpu.make_async_copy(seqids_hbm.at[addr_v.at[j]], ids_v.at[j],
                                  sem1).wait()
            return carry

        lax.fori_loop(0, SEQ_CH, g1w, 0)

        sbase = wid * SEQ_PW

        def wb_dst(j):
            return padded_hbm.at[pl.ds(sbase + j * 128, 128)]

        for p in range(NBUF - 1):
            pltpu.async_copy(dinT_hbm.at[ids_v.at[p]], rows_v.at[p], semg[p])

        def g2(j, carry):
            for pp in range(NBUF):
                @pl.when(lax.rem(j, NBUF) == pp)
                def _(pp=pp):
                    qq = (pp + NBUF - 1) % NBUF
                    pltpu.make_async_copy(dinT_hbm.at[ids_v.at[j]],
                                          rows_v.at[pp], semg[pp]).wait()

                    @pl.when(j + NBUF - 1 < SEQ_CH)
                    def _():
                        @pl.when(j >= 1)
                        def _():
                            pltpu.make_async_copy(rows_v.at[qq],
                                                  wb_dst(j - 1),
                                                  semw[qq]).wait()

                        pltpu.async_copy(dinT_hbm.at[ids_v.at[j + NBUF - 1]],
                                         rows_v.at[qq], semg[qq])

                    pltpu.async_copy(rows_v.at[pp], wb_dst(j), semw[pp])
            return carry

        lax.fori_loop(0, SEQ_CH, g2, 0)
        for pp in range(NBUF):
            pltpu.make_async_copy(rows_v.at[pp], wb_dst(0), semw[pp]).wait()

        pltpu.sync_copy(oidx_hbm.at[wid], smidx_v.at[0])
        pltpu.sync_copy(tidx_hbm.at[wid], smidx_v.at[1])
        bbase = wid * BPW
        pltpu.async_copy(dinT_hbm.at[smidx_v.at[0]], srows_v, sem1).wait()
        pltpu.sync_copy(srows_v, oemb_hbm.at[pl.ds(bbase, BPW)])
        pltpu.async_copy(dinT_hbm.at[smidx_v.at[1]], srows_v, sem1).wait()
        pltpu.sync_copy(srows_v, temb_hbm.at[pl.ds(bbase, BPW)])

    return k(addr, seq_flat_ids, din_table, oth_idx, tgt_idx)


def _sc_feat_gather(deep_table, wide_table, deep_idx, wide_idx,
                    NFB, DD, WD, FTAIL):
    F_CH = deep_idx.shape[1]
    F_PW = NFB // NW

    mesh = plsc.VectorSubcoreMesh(core_axis_name="c", subcore_axis_name="s")

    @functools.partial(
        pl.kernel,
        out_type=(
            jax.ShapeDtypeStruct((NFB, DD), jnp.float32),
            jax.ShapeDtypeStruct((NFB, WD), jnp.float32),
        ),
        mesh=mesh,
        compiler_params=pltpu.CompilerParams(use_tc_tiling_on_sc=False),
        scratch_types=[
            pltpu.VMEM((F_CH, 128), jnp.int32),
            pltpu.VMEM((2, 128, DD), jnp.float32),
            pltpu.VMEM((128, WD), jnp.float32),
            pltpu.SemaphoreType.DMA,
            pltpu.SemaphoreType.DMA,
            pltpu.SemaphoreType.DMA,
        ],
    )
    def k(deepT_hbm, wideT_hbm, didx_hbm, widx_hbm,
          drows_hbm, wrows_hbm,
          fidx_v, rows_v, wrows_v, semg, semw, sem1):
        c = lax.axis_index("c")
        s = lax.axis_index("s")
        wid = s * NC + c
        fbase = wid * F_PW

        pltpu.sync_copy(didx_hbm.at[wid], fidx_v)
        pltpu.async_copy(deepT_hbm.at[fidx_v.at[0]], rows_v.at[0], semg)

        def g3(j, carry):
            for pp in range(2):
                @pl.when(lax.rem(j, 2) == pp)
                def _(pp=pp):
                    pltpu.make_async_copy(deepT_hbm.at[fidx_v.at[j]],
                                          rows_v.at[pp], semg).wait()

                    @pl.when(j + 1 < F_CH)
                    def _():
                        @pl.when(j >= 1)
                        def _():
                            pltpu.make_async_copy(
                                rows_v.at[1 - pp],
                                drows_hbm.at[pl.ds(fbase, 128)], semw).wait()

                        pltpu.async_copy(deepT_hbm.at[fidx_v.at[j + 1]],
                                         rows_v.at[1 - pp], semg)

                    @pl.when(j < F_CH - 1)
                    def _():
                        pltpu.async_copy(
                            rows_v.at[pp],
                            drows_hbm.at[pl.ds(fbase + j * 128, 128)], semw)

                    @pl.when(j == F_CH - 1)
                    def _():
                        pltpu.async_copy(
                            rows_v.at[pp].at[pl.ds(0, FTAIL)],
                            drows_hbm.at[pl.ds(fbase + j * 128, FTAIL)],
                            semw)

            return carry

        lax.fori_loop(0, F_CH, g3, 0)
        pltpu.make_async_copy(rows_v.at[0],
                              drows_hbm.at[pl.ds(fbase, 128)], semw).wait()
        pltpu.make_async_copy(rows_v.at[0].at[pl.ds(0, FTAIL)],
                              drows_hbm.at[pl.ds(fbase, FTAIL)], semw).wait()

        pltpu.sync_copy(widx_hbm.at[wid], fidx_v)

        def g4(j, carry):
            pltpu.async_copy(wideT_hbm.at[fidx_v.at[j]], wrows_v,
                             sem1).wait()
            pltpu.sync_copy(wrows_v,
                            wrows_hbm.at[pl.ds(fbase + j * 128, 128)])
            return carry

        lax.fori_loop(0, F_CH - 1, g4, 0)
        pltpu.async_copy(wideT_hbm.at[fidx_v.at[F_CH - 1]], wrows_v,
                         sem1).wait()
        pltpu.sync_copy(
            wrows_v.at[pl.ds(0, FTAIL)],
            wrows_hbm.at[pl.ds(fbase + (F_CH - 1) * 128, FTAIL)])

    return k(deep_table, wide_table, deep_idx, wide_idx)


def _dice_v(x, st, n, alpha):
    mean = st[0:1, :] * (1.0 / n)
    var = st[1:2, :] * (1.0 / n) - mean * mean
    xn = (x - mean) * lax.rsqrt(var + 1e-5)
    p = jax.nn.sigmoid(xn)
    return x * (p + (1.0 - p) * alpha)


def _p123_body(BB, LMAX, D, GA, n1,
               pad_ref, tgt_ref, len_ref,
               wA_ref, wB_ref, wC_ref, b1_ref, a1_ref,
               w2_ref, b2_ref, a2_ref, wo_ref, bo_ref,
               pooled_ref,
               h2_v, st1_v, st2_v):
    i = pl.program_id(0)
    R = BB * LMAX

    @pl.when(i == 0)
    def _():
        st1_v[...] = jnp.zeros_like(st1_v)
        st2_v[...] = jnp.zeros_like(st2_v)

    def h1_block():
        p3 = pad_ref[...].reshape(BB, LMAX, D)
        liota = lax.broadcasted_iota(jnp.int32, (BB, LMAX, D), 1)
        liota = liota.astype(jnp.float32)
        len3 = len_ref[...].reshape(BB, 1, 1)
        p3 = jnp.where(liota < len3, p3, 0.0)
        t3 = jnp.broadcast_to(tgt_ref[...].reshape(BB, 1, D), (BB, LMAX, D))
        pb = p3.reshape(R, D).astype(jnp.bfloat16)
        tpb = (t3 * p3).reshape(R, D).astype(jnp.bfloat16)
        h = jnp.dot(pb, wB_ref[...], preferred_element_type=jnp.float32)
        h += jnp.dot(tpb, wC_ref[...], preferred_element_type=jnp.float32)
        ta = jnp.dot(tgt_ref[...].astype(jnp.bfloat16), wA_ref[...],
                     preferred_element_type=jnp.float32)
        H = ta.shape[1]
        h = (h.reshape(BB, LMAX, H) + ta.reshape(BB, 1, H)).reshape(R, H)
        return h + b1_ref[...]

    @pl.when(i < GA)
    def _():
        h = h1_block()
        st1_v[0:1, :] += jnp.sum(h, axis=0, keepdims=True)
        st1_v[1:2, :] += jnp.sum(h * h, axis=0, keepdims=True)

    @pl.when((i >= GA) & (i < 2 * GA))
    def _():
        j = i - GA
        x = _dice_v(h1_block(), st1_v[...], n1, a1_ref[...])
        h2 = jnp.dot(x.astype(jnp.bfloat16), w2_ref[...],
                     preferred_element_type=jnp.float32)
        h2 = h2 + b2_ref[...]
        h2_v[pl.ds(j * R, R), :] = h2.astype(jnp.bfloat16)
        st2_v[0:1, :] += jnp.sum(h2, axis=0, keepdims=True)
        st2_v[1:2, :] += jnp.sum(h2 * h2, axis=0, keepdims=True)

    @pl.when(i >= 2 * GA)
    def _():
        k = i - 2 * GA
        x = h2_v[pl.ds(k * R, R), :].astype(jnp.float32)
        x = _dice_v(x, st2_v[...], n1, a2_ref[...])
        s = jnp.sum(x * wo_ref[...], axis=1, keepdims=True) + bo_ref[...]
        s3 = s.reshape(BB, LMAX, 1)
        liota = lax.broadcasted_iota(jnp.int32, (BB, LMAX, 1), 1)
        liota = liota.astype(jnp.float32)
        len3 = len_ref[...].reshape(BB, 1, 1)
        s3 = jnp.where(liota < len3, s3, 0.0)
        p3 = pad_ref[...].reshape(BB, LMAX, D)
        pooled_ref[...] = jnp.sum(p3 * s3, axis=1)


def _bn_in(x, n):
    mean = jnp.sum(x, axis=0, keepdims=True) * (1.0 / n)
    d = x - mean
    var = jnp.sum(d * d, axis=0, keepdims=True) * (1.0 / n)
    return d * lax.rsqrt(var + 1e-5)


def _dice_in(x, n, alpha):
    xn = _bn_in(x, n)
    p = jax.nn.sigmoid(xn)
    return x * (p + (1.0 - p) * alpha)


def _p4_body(B,
             oth_ref, pool_ref, tgt_ref,
             mW1_ref, mb1_ref, ma1_ref, mW2_ref, mb2_ref, ma2_ref,
             mWo_ref, mbo_ref,
             wide_ref, lrW_ref, lrb_ref,
             deep_ref, dW1_ref, db1_ref, dW2_ref, db2_ref, dWo_ref, dbo_ref,
             out_ref):
    n = float(B)
    bf = jnp.bfloat16
    emb = jnp.concatenate(
        [oth_ref[...], pool_ref[...], tgt_ref[...]], axis=1)
    x = jnp.dot(emb.astype(bf), mW1_ref[...],
                preferred_element_type=jnp.float32)
    x = _dice_in(x + mb1_ref[...], n, ma1_ref[...])
    x = jnp.dot(x.astype(bf), mW2_ref[...],
                preferred_element_type=jnp.float32)
    x = _dice_in(x + mb2_ref[...], n, ma2_ref[...])
    din = jnp.sum(x * mWo_ref[...], axis=1, keepdims=True) + mbo_ref[...]

    lr = jnp.sum(wide_ref[...] * lrW_ref[...], axis=1, keepdims=True)
    lr = lr + lrb_ref[...]

    d = jnp.dot(deep_ref[...].astype(bf), dW1_ref[...],
                preferred_element_type=jnp.float32)
    d = jax.nn.relu(_bn_in(d + db1_ref[...], n))
    d = jnp.dot(d.astype(bf), dW2_ref[...],
                preferred_element_type=jnp.float32)
    d = jax.nn.relu(_bn_in(d + db2_ref[...], n))
    deep = jnp.sum(d * dWo_ref[...], axis=1, keepdims=True) + dbo_ref[...]

    out_ref[...] = jax.nn.sigmoid(din + lr + deep)


def kernel(other_ids, seq_flat_ids, cu_seqlens, target_ids, wide_ids, deep_ids,
           din_table, aW1, ab1, aa1, aW2, ab2, aa2, aWo, abo,
           mW1, mb1, ma1, mW2, mb2, ma2, mWo, mbo,
           wide_table, lrW, lrb, deep_table, dW1, db1, dW2, db2, dWo, dbo):
    B = other_ids.shape[0]
    T = seq_flat_ids.shape[0]
    D = din_table.shape[1]
    WD = wide_table.shape[1]
    DD = deep_table.shape[1]
    NF = wide_ids.shape[1]
    LMAX = 200
    BL = B * LMAX

    cu = cu_seqlens.astype(jnp.int32)
    lengths = cu[1:] - cu[:-1]
    addr = cu[:-1, None] + jnp.arange(LMAX, dtype=jnp.int32)[None, :]
    addr = jnp.minimum(addr, T - 1).reshape(NW, BL // NW // 128, 128)

    nf_flat = B * NF
    F_PW = nf_flat // NW
    F_CH = -(-F_PW // 128)
    FTAIL = F_PW - (F_CH - 1) * 128
    zpad = jnp.zeros((NW, F_CH * 128 - F_PW), jnp.int32)
    didx = jnp.concatenate(
        [deep_ids.reshape(NW, F_PW).astype(jnp.int32), zpad], axis=1)
    widx = jnp.concatenate(
        [wide_ids.reshape(NW, F_PW).astype(jnp.int32), zpad], axis=1)
    didx = didx.reshape(NW, F_CH, 128)
    widx = widx.reshape(NW, F_CH, 128)
    oidx = other_ids.astype(jnp.int32).reshape(NW, B // NW)
    tidx = target_ids.astype(jnp.int32).reshape(NW, B // NW)

    padded, oth_emb, tgt_emb = _sc_seq_gather(
        addr, seq_flat_ids.astype(jnp.int32), din_table, oidx, tidx, BL, D)
    drows, wrows = _sc_feat_gather(
        deep_table, wide_table, didx, widx, nf_flat, DD, WD, FTAIL)

    lenf = lengths.astype(jnp.float32).reshape(B, 1)

    bf = jnp.bfloat16
    wA = (aW1[:D] + aW1[2 * D:3 * D]).astype(bf)
    wB = (aW1[D:2 * D] - aW1[2 * D:3 * D]).astype(bf)
    wC = aW1[3 * D:].astype(bf)

    BB = 8
    GA = B // BB
    R = BB * LMAX
    H1 = aW1.shape[1]
    H2 = aW2.shape[1]

    def pad_map(i):
        return (lax.rem(i, GA), 0)

    tgt_map = pad_map
    cst = lambda i: (0, 0)
    pool_map = lambda i: (jnp.maximum(i - 2 * GA, 0), 0)
    pooled = pl.pallas_call(
        functools.partial(_p123_body, BB, LMAX, D, GA, float(BL)),
        grid=(3 * GA,),
        in_specs=[
            pl.BlockSpec((R, D), pad_map),
            pl.BlockSpec((BB, D), tgt_map),
            pl.BlockSpec((BB, 1), pad_map),
            pl.BlockSpec((D, H1), cst),
            pl.BlockSpec((D, H1), cst),
            pl.BlockSpec((D, H1), cst),
            pl.BlockSpec((1, H1), cst),
            pl.BlockSpec((1, H1), cst),
            pl.BlockSpec((H1, H2), cst),
            pl.BlockSpec((1, H2), cst),
            pl.BlockSpec((1, H2), cst),
            pl.BlockSpec((1, H2), cst),
            pl.BlockSpec((1, 1), cst),
        ],
        out_specs=pl.BlockSpec((BB, D), pool_map),
        out_shape=jax.ShapeDtypeStruct((B, D), jnp.float32),
        scratch_shapes=[
            pltpu.VMEM((BL, H2), bf),
            pltpu.VMEM((2, H1), jnp.float32),
            pltpu.VMEM((2, H2), jnp.float32),
        ],
        compiler_params=pltpu.CompilerParams(
            vmem_limit_bytes=60 * 1024 * 1024),
    )(padded, tgt_emb, lenf,
      wA, wB, wC, ab1.reshape(1, H1), aa1.reshape(1, H1),
      aW2.astype(bf), ab2.reshape(1, H2), aa2.reshape(1, H2),
      aWo.reshape(1, H2), abo.reshape(1, 1))

    wide_flat = wrows.reshape(B, NF * WD)
    deep_flat = drows.reshape(B, NF * DD)
    M1 = mW1.shape[1]
    M2 = mW2.shape[1]
    DH1 = dW1.shape[1]
    DH2 = dW2.shape[1]
    full = lambda a, b: pl.BlockSpec((a, b), lambda: (0, 0))
    out = pl.pallas_call(
        functools.partial(_p4_body, B),
        in_specs=[
            full(B, D), full(B, D), full(B, D),
            full(3 * D, M1), full(1, M1), full(1, M1),
            full(M1, M2), full(1, M2), full(1, M2),
            full(1, M2), full(1, 1),
            full(B, NF * WD), full(1, NF * WD), full(1, 1),
            full(B, NF * DD),
            full(NF * DD, DH1), full(1, DH1),
            full(DH1, DH2), full(1, DH2), full(1, DH2), full(1, 1),
        ],
        out_specs=full(B, 1),
        out_shape=jax.ShapeDtypeStruct((B, 1), jnp.float32),
        compiler_params=pltpu.CompilerParams(
            vmem_limit_bytes=100 * 1024 * 1024),
    )(oth_emb, pooled, tgt_emb,
      mW1.astype(bf), mb1.reshape(1, M1), ma1.reshape(1, M1),
      mW2.astype(bf), mb2.reshape(1, M2), ma2.reshape(1, M2),
      mWo.reshape(1, M2), mbo.reshape(1, 1),
      wide_flat, lrW.reshape(1, NF * WD), lrb.reshape(1, 1),
      deep_flat, dW1.astype(bf), db1.reshape(1, DH1),
      dW2.astype(bf), db2.reshape(1, DH2), dWo.reshape(1, DH2),
      dbo.reshape(1, 1))
    return out


# speedup vs baseline: 15.1094x; 1.1603x over previous
import functools

import jax
import jax.numpy as jnp
from jax import lax
from jax.experimental import pallas as pl
from jax.experimental.pallas import tpu as pltpu
from jax.experimental.pallas import tpu_sc as plsc

NC = 2
NS = 16
NW = NC * NS
NBUF = 4


def _sc_seq_gather(addr, seq_flat_ids, din_table, oth_idx, tgt_idx, BL, D):
    SEQ_CH = addr.shape[1]
    BPW = oth_idx.shape[1]
    B = NW * BPW
    SEQ_PW = SEQ_CH * 128

    mesh = plsc.VectorSubcoreMesh(core_axis_name="c", subcore_axis_name="s")

    @functools.partial(
        pl.kernel,
        out_type=(
            jax.ShapeDtypeStruct((BL, D), jnp.float32),
            jax.ShapeDtypeStruct((B, D), jnp.float32),
            jax.ShapeDtypeStruct((B, D), jnp.float32),
        ),
        mesh=mesh,
        compiler_params=pltpu.CompilerParams(use_tc_tiling_on_sc=False),
        scratch_types=[
            pltpu.VMEM((SEQ_CH, 128), jnp.int32),
            pltpu.VMEM((SEQ_CH, 128), jnp.int32),
            pltpu.VMEM((2, BPW), jnp.int32),
            pltpu.VMEM((NBUF, 128, D), jnp.float32),
            pltpu.VMEM((BPW, D), jnp.float32),
        ] + [pltpu.SemaphoreType.DMA] * (2 * NBUF + 1),
    )
    def k(addr_hbm, seqids_hbm, dinT_hbm, oidx_hbm, tidx_hbm,
          padded_hbm, oemb_hbm, temb_hbm,
          addr_v, ids_v, smidx_v, rows_v, srows_v, *sems):
        semg = sems[:NBUF]
        semw = sems[NBUF:2 * NBUF]
        sem1 = sems[2 * NBUF]
        c = lax.axis_index("c")
        s = lax.axis_index("s")
        wid = s * NC + c

        pltpu.sync_copy(addr_hbm.at[wid], addr_v)

        def g1(j, carry):
            pltpu.async_copy(seqids_hbm.at[addr_v.at[j]], ids_v.at[j], sem1)
            return carry

        lax.fori_loop(0, SEQ_CH, g1, 0)

        def g1w(j, carry):
            pltpu.make_async_copy(seqids_hbm.at[addr_v.at[j]], ids_v.at[j],
                                  sem1).wait()
            return carry

        lax.fori_loop(0, SEQ_CH, g1w, 0)

        sbase = wid * SEQ_PW

        def wb_dst(j):
            return padded_hbm.at[pl.ds(sbase + j * 128, 128)]

        for p in range(NBUF - 1):
            pltpu.async_copy(dinT_hbm.at[ids_v.at[p]], rows_v.at[p], semg[p])

        def g2(j, carry):
            for pp in range(NBUF):
                @pl.when(lax.rem(j, NBUF) == pp)
                def _(pp=pp):
                    qq = (pp + NBUF - 1) % NBUF
                    pltpu.make_async_copy(dinT_hbm.at[ids_v.at[j]],
                                          rows_v.at[pp], semg[pp]).wait()

                    @pl.when(j + NBUF - 1 < SEQ_CH)
                    def _():
                        @pl.when(j >= 1)
                        def _():
                            pltpu.make_async_copy(rows_v.at[qq],
                                                  wb_dst(j - 1),
                                                  semw[qq]).wait()

                        pltpu.async_copy(dinT_hbm.at[ids_v.at[j + NBUF - 1]],
                                         rows_v.at[qq], semg[qq])

                    pltpu.async_copy(rows_v.at[pp], wb_dst(j), semw[pp])
            return carry

        lax.fori_loop(0, SEQ_CH, g2, 0)
        for pp in range(NBUF):
            pltpu.make_async_copy(rows_v.at[pp], wb_dst(0), semw[pp]).wait()

        pltpu.sync_copy(oidx_hbm.at[wid], smidx_v.at[0])
        pltpu.sync_copy(tidx_hbm.at[wid], smidx_v.at[1])
        bbase = wid * BPW
        pltpu.async_copy(dinT_hbm.at[smidx_v.at[0]], srows_v, sem1).wait()
        pltpu.sync_copy(srows_v, oemb_hbm.at[pl.ds(bbase, BPW)])
        pltpu.async_copy(dinT_hbm.at[smidx_v.at[1]], srows_v, sem1).wait()
        pltpu.sync_copy(srows_v, temb_hbm.at[pl.ds(bbase, BPW)])

    return k(addr, seq_flat_ids, din_table, oth_idx, tgt_idx)


def _sc_feat_gather(deep_table, wide_table, deep_idx, wide_idx,
                    NFB, DD, WD, FTAIL):
    F_CH = deep_idx.shape[1]
    F_PW = NFB // NW

    mesh = plsc.VectorSubcoreMesh(core_axis_name="c", subcore_axis_name="s")

    @functools.partial(
        pl.kernel,
        out_type=(
            jax.ShapeDtypeStruct((NFB, DD), jnp.float32),
            jax.ShapeDtypeStruct((NFB, WD), jnp.float32),
        ),
        mesh=mesh,
        compiler_params=pltpu.CompilerParams(use_tc_tiling_on_sc=False),
        scratch_types=[
            pltpu.VMEM((F_CH, 128), jnp.int32),
            pltpu.VMEM((2, 128, DD), jnp.float32),
            pltpu.VMEM((128, WD), jnp.float32),
            pltpu.SemaphoreType.DMA,
            pltpu.SemaphoreType.DMA,
            pltpu.SemaphoreType.DMA,
        ],
    )
    def k(deepT_hbm, wideT_hbm, didx_hbm, widx_hbm,
          drows_hbm, wrows_hbm,
          fidx_v, rows_v, wrows_v, semg, semw, sem1):
        c = lax.axis_index("c")
        s = lax.axis_index("s")
        wid = s * NC + c
        fbase = wid * F_PW

        pltpu.sync_copy(didx_hbm.at[wid], fidx_v)
        pltpu.async_copy(deepT_hbm.at[fidx_v.at[0]], rows_v.at[0], semg)

        def g3(j, carry):
            for pp in range(2):
                @pl.when(lax.rem(j, 2) == pp)
                def _(pp=pp):
                    pltpu.make_async_copy(deepT_hbm.at[fidx_v.at[j]],
                                          rows_v.at[pp], semg).wait()

                    @pl.when(j + 1 < F_CH)
                    def _():
                        @pl.when(j >= 1)
                        def _():
                            pltpu.make_async_copy(
                                rows_v.at[1 - pp],
                                drows_hbm.at[pl.ds(fbase, 128)], semw).wait()

                        pltpu.async_copy(deepT_hbm.at[fidx_v.at[j + 1]],
                                         rows_v.at[1 - pp], semg)

                    @pl.when(j < F_CH - 1)
                    def _():
                        pltpu.async_copy(
                            rows_v.at[pp],
                            drows_hbm.at[pl.ds(fbase + j * 128, 128)], semw)

                    @pl.when(j == F_CH - 1)
                    def _():
                        pltpu.async_copy(
                            rows_v.at[pp].at[pl.ds(0, FTAIL)],
                            drows_hbm.at[pl.ds(fbase + j * 128, FTAIL)],
                            semw)

            return carry

        lax.fori_loop(0, F_CH, g3, 0)
        pltpu.make_async_copy(rows_v.at[0],
                              drows_hbm.at[pl.ds(fbase, 128)], semw).wait()
        pltpu.make_async_copy(rows_v.at[0].at[pl.ds(0, FTAIL)],
                              drows_hbm.at[pl.ds(fbase, FTAIL)], semw).wait()

        pltpu.sync_copy(widx_hbm.at[wid], fidx_v)

        def g4(j, carry):
            pltpu.async_copy(wideT_hbm.at[fidx_v.at[j]], wrows_v,
                             sem1).wait()
            pltpu.sync_copy(wrows_v,
                            wrows_hbm.at[pl.ds(fbase + j * 128, 128)])
            return carry

        lax.fori_loop(0, F_CH - 1, g4, 0)
        pltpu.async_copy(wideT_hbm.at[fidx_v.at[F_CH - 1]], wrows_v,
                         sem1).wait()
        pltpu.sync_copy(
            wrows_v.at[pl.ds(0, FTAIL)],
            wrows_hbm.at[pl.ds(fbase + (F_CH - 1) * 128, FTAIL)])

    return k(deep_table, wide_table, deep_idx, wide_idx)


def _dice_v(x, st, n, alpha):
    mean = st[0:1, :] * (1.0 / n)
    var = st[1:2, :] * (1.0 / n) - mean * mean
    xn = (x - mean) * lax.rsqrt(var + 1e-5)
    p = jax.nn.sigmoid(xn)
    return x * (p + (1.0 - p) * alpha)


def _p123_body(BB, LMAX, D, GA, n1,
               pad_ref, tgt_ref, len_ref,
               wA_ref, wB_ref, wC_ref, b1_ref, a1_ref,
               w2_ref, b2_ref, a2_ref, wo_ref, bo_ref,
               pooled_ref,
               h2_v, st1_v, st2_v):
    i = pl.program_id(0)
    R = BB * LMAX

    @pl.when(i == 0)
    def _():
        st1_v[...] = jnp.zeros_like(st1_v)
        st2_v[...] = jnp.zeros_like(st2_v)

    def h1_block():
        p3 = pad_ref[...].reshape(BB, LMAX, D)
        liota = lax.broadcasted_iota(jnp.int32, (BB, LMAX, D), 1)
        liota = liota.astype(jnp.float32)
        len3 = len_ref[...].reshape(BB, 1, 1)
        p3 = jnp.where(liota < len3, p3, 0.0)
        t3 = jnp.broadcast_to(tgt_ref[...].reshape(BB, 1, D), (BB, LMAX, D))
        pb = p3.reshape(R, D).astype(jnp.bfloat16)
        tpb = (t3 * p3).reshape(R, D).astype(jnp.bfloat16)
        h = jnp.dot(pb, wB_ref[...], preferred_element_type=jnp.float32)
        h += jnp.dot(tpb, wC_ref[...], preferred_element_type=jnp.float32)
        ta = jnp.dot(tgt_ref[...].astype(jnp.bfloat16), wA_ref[...],
                     preferred_element_type=jnp.float32)
        H = ta.shape[1]
        h = (h.reshape(BB, LMAX, H) + ta.reshape(BB, 1, H)).reshape(R, H)
        return h + b1_ref[...]

    @pl.when(i < GA)
    def _():
        h = h1_block()
        st1_v[0:1, :] += jnp.sum(h, axis=0, keepdims=True)
        st1_v[1:2, :] += jnp.sum(h * h, axis=0, keepdims=True)

    @pl.when((i >= GA) & (i < 2 * GA))
    def _():
        j = i - GA
        x = _dice_v(h1_block(), st1_v[...], n1, a1_ref[...])
        h2 = jnp.dot(x.astype(jnp.bfloat16), w2_ref[...],
                     preferred_element_type=jnp.float32)
        h2 = h2 + b2_ref[...]
        h2_v[pl.ds(j * R, R), :] = h2.astype(jnp.bfloat16)
        st2_v[0:1, :] += jnp.sum(h2, axis=0, keepdims=True)
        st2_v[1:2, :] += jnp.sum(h2 * h2, axis=0, keepdims=True)

    @pl.when(i >= 2 * GA)
    def _():
        k = i - 2 * GA
        x = h2_v[pl.ds(k * R, R), :].astype(jnp.float32)
        x = _dice_v(x, st2_v[...], n1, a2_ref[...])
        s = jnp.sum(x * wo_ref[...], axis=1, keepdims=True) + bo_ref[...]
        s3 = s.reshape(BB, LMAX, 1)
        liota = lax.broadcasted_iota(jnp.int32, (BB, LMAX, 1), 1)
        liota = liota.astype(jnp.float32)
        len3 = len_ref[...].reshape(BB, 1, 1)
        s3 = jnp.where(liota < len3, s3, 0.0)
        p3 = pad_ref[...].reshape(BB, LMAX, D)
        pooled_ref[...] = jnp.sum(p3 * s3, axis=1)


def _bn_in(x, n):
    mean = jnp.sum(x, axis=0, keepdims=True) * (1.0 / n)
    d = x - mean
    var = jnp.sum(d * d, axis=0, keepdims=True) * (1.0 / n)
    return d * lax.rsqrt(var + 1e-5)


def _dice_in(x, n, alpha):
    xn = _bn_in(x, n)
    p = jax.nn.sigmoid(xn)
    return x * (p + (1.0 - p) * alpha)


def _p4_body(B,
             oth_ref, pool_ref, tgt_ref,
             mW1_ref, mb1_ref, ma1_ref, mW2_ref, mb2_ref, ma2_ref,
             mWo_ref, mbo_ref,
             wide_ref, lrW_ref, lrb_ref,
             deep_ref, dW1_ref, db1_ref, dW2_ref, db2_ref, dWo_ref, dbo_ref,
             out_ref):
    n = float(B)
    bf = jnp.bfloat16
    emb = jnp.concatenate(
        [oth_ref[...], pool_ref[...], tgt_ref[...]], axis=1)
    x = jnp.dot(emb.astype(bf), mW1_ref[...],
                preferred_element_type=jnp.float32)
    x = _dice_in(x + mb1_ref[...], n, ma1_ref[...])
    x = jnp.dot(x.astype(bf), mW2_ref[...],
                preferred_element_type=jnp.float32)
    x = _dice_in(x + mb2_ref[...], n, ma2_ref[...])
    din = jnp.sum(x * mWo_ref[...], axis=1, keepdims=True) + mbo_ref[...]

    lr = jnp.sum(wide_ref[...] * lrW_ref[...], axis=1, keepdims=True)
    lr = lr + lrb_ref[...]

    d = jnp.dot(deep_ref[...].astype(bf), dW1_ref[...],
                preferred_element_type=jnp.float32)
    d = jax.nn.relu(_bn_in(d + db1_ref[...], n))
    d = jnp.dot(d.astype(bf), dW2_ref[...],
                preferred_element_type=jnp.float32)
    d = jax.nn.relu(_bn_in(d + db2_ref[...], n))
    deep = jnp.sum(d * dWo_ref[...], axis=1, keepdims=True) + dbo_ref[...]

    out_ref[...] = jax.nn.sigmoid(din + lr + deep)


def kernel(other_ids, seq_flat_ids, cu_seqlens, target_ids, wide_ids, deep_ids,
           din_table, aW1, ab1, aa1, aW2, ab2, aa2, aWo, abo,
           mW1, mb1, ma1, mW2, mb2, ma2, mWo, mbo,
           wide_table, lrW, lrb, deep_table, dW1, db1, dW2, db2, dWo, dbo):
    B = other_ids.shape[0]
    T = seq_flat_ids.shape[0]
    D = din_table.shape[1]
    WD = wide_table.shape[1]
    DD = deep_table.shape[1]
    NF = wide_ids.shape[1]
    LMAX = 200
    BL = B * LMAX

    cu = cu_seqlens.astype(jnp.int32)
    lengths = cu[1:] - cu[:-1]
    addr = cu[:-1, None] + jnp.arange(LMAX, dtype=jnp.int32)[None, :]
    addr = jnp.minimum(addr, T - 1).reshape(NW, BL // NW // 128, 128)

    nf_flat = B * NF
    F_PW = nf_flat // NW
    F_CH = -(-F_PW // 128)
    FTAIL = F_PW - (F_CH - 1) * 128
    zpad = jnp.zeros((NW, F_CH * 128 - F_PW), jnp.int32)
    didx = jnp.concatenate(
        [deep_ids.reshape(NW, F_PW).astype(jnp.int32), zpad], axis=1)
    widx = jnp.concatenate(
        [wide_ids.reshape(NW, F_PW).astype(jnp.int32), zpad], axis=1)
    didx = didx.reshape(NW, F_CH, 128)
    widx = widx.reshape(NW, F_CH, 128)
    oidx = other_ids.astype(jnp.int32).reshape(NW, B // NW)
    tidx = target_ids.astype(jnp.int32).reshape(NW, B // NW)

    padded, oth_emb, tgt_emb = _sc_seq_gather(
        addr, seq_flat_ids.astype(jnp.int32), din_table, oidx, tidx, BL, D)
    drows, wrows = _sc_feat_gather(
        deep_table, wide_table, didx, widx, nf_flat, DD, WD, FTAIL)

    lenf = lengths.astype(jnp.float32).reshape(B, 1)

    bf = jnp.bfloat16
    wA = (aW1[:D] + aW1[2 * D:3 * D]).astype(bf)
    wB = (aW1[D:2 * D] - aW1[2 * D:3 * D]).astype(bf)
    wC = aW1[3 * D:].astype(bf)

    BB = 16
    GA = B // BB
    R = BB * LMAX
    H1 = aW1.shape[1]
    H2 = aW2.shape[1]

    def pad_map(i):
        return (lax.rem(i, GA), 0)

    tgt_map = pad_map
    cst = lambda i: (0, 0)
    pool_map = lambda i: (jnp.maximum(i - 2 * GA, 0), 0)
    pooled = pl.pallas_call(
        functools.partial(_p123_body, BB, LMAX, D, GA, float(BL)),
        grid=(3 * GA,),
        in_specs=[
            pl.BlockSpec((R, D), pad_map),
            pl.BlockSpec((BB, D), tgt_map),
            pl.BlockSpec((BB, 1), pad_map),
            pl.BlockSpec((D, H1), cst),
            pl.BlockSpec((D, H1), cst),
            pl.BlockSpec((D, H1), cst),
            pl.BlockSpec((1, H1), cst),
            pl.BlockSpec((1, H1), cst),
            pl.BlockSpec((H1, H2), cst),
            pl.BlockSpec((1, H2), cst),
            pl.BlockSpec((1, H2), cst),
            pl.BlockSpec((1, H2), cst),
            pl.BlockSpec((1, 1), cst),
        ],
        out_specs=pl.BlockSpec((BB, D), pool_map),
        out_shape=jax.ShapeDtypeStruct((B, D), jnp.float32),
        scratch_shapes=[
            pltpu.VMEM((BL, H2), bf),
            pltpu.VMEM((2, H1), jnp.float32),
            pltpu.VMEM((2, H2), jnp.float32),
        ],
        compiler_params=pltpu.CompilerParams(
            vmem_limit_bytes=60 * 1024 * 1024),
    )(padded, tgt_emb, lenf,
      wA, wB, wC, ab1.reshape(1, H1), aa1.reshape(1, H1),
      aW2.astype(bf), ab2.reshape(1, H2), aa2.reshape(1, H2),
      aWo.reshape(1, H2), abo.reshape(1, 1))

    wide_flat = wrows.reshape(B, NF * WD)
    deep_flat = drows.reshape(B, NF * DD)
    M1 = mW1.shape[1]
    M2 = mW2.shape[1]
    DH1 = dW1.shape[1]
    DH2 = dW2.shape[1]
    full = lambda a, b: pl.BlockSpec((a, b), lambda: (0, 0))
    out = pl.pallas_call(
        functools.partial(_p4_body, B),
        in_specs=[
            full(B, D), full(B, D), full(B, D),
            full(3 * D, M1), full(1, M1), full(1, M1),
            full(M1, M2), full(1, M2), full(1, M2),
            full(1, M2), full(1, 1),
            full(B, NF * WD), full(1, NF * WD), full(1, 1),
            full(B, NF * DD),
            full(NF * DD, DH1), full(1, DH1),
            full(DH1, DH2), full(1, DH2), full(1, DH2), full(1, 1),
        ],
        out_specs=full(B, 1),
        out_shape=jax.ShapeDtypeStruct((B, 1), jnp.float32),
        compiler_params=pltpu.CompilerParams(
            vmem_limit_bytes=100 * 1024 * 1024),
    )(oth_emb, pooled, tgt_emb,
      mW1.astype(bf), mb1.reshape(1, M1), ma1.reshape(1, M1),
      mW2.astype(bf), mb2.reshape(1, M2), ma2.reshape(1, M2),
      mWo.reshape(1, M2), mbo.reshape(1, 1),
      wide_flat, lrW.reshape(1, NF * WD), lrb.reshape(1, 1),
      deep_flat, dW1.astype(bf), db1.reshape(1, DH1),
      dW2.astype(bf), db2.reshape(1, DH2), dWo.reshape(1, DH2),
      dbo.reshape(1, 1))
    return out
